# baseline (speedup 1.0000x reference)
import functools

import jax
import jax.numpy as jnp
from jax import lax
from jax.experimental import pallas as pl
from jax.experimental.pallas import tpu as pltpu
from jax.experimental.pallas import tpu_sc as plsc


_NUM_CORES = 2
_NUM_SUBCORES = 16
_LANES = 16


@functools.lru_cache(maxsize=None)
def _build_adj_fn(n_nodes: int, n_edges: int):
    n_workers = _NUM_CORES * _NUM_SUBCORES
    rows = n_nodes // n_workers
    half = rows // 2
    nhalf = n_nodes // 2
    wcols = n_nodes // 2
    ech = 16384
    n_chunks = n_edges // ech
    zunroll = 8
    eunroll = 4

    mesh = plsc.VectorSubcoreMesh(core_axis_name="c", subcore_axis_name="s")

    @functools.partial(
        pl.kernel,
        mesh=mesh,
        compiler_params=pltpu.CompilerParams(needs_layout_passes=False),
        out_type=jax.ShapeDtypeStruct((n_nodes, wcols), jnp.int32),
        scratch_types=[
            pltpu.VMEM((rows + 1, wcols), jnp.int32),
            pltpu.VMEM((ech,), jnp.int32),
            pltpu.VMEM((ech,), jnp.int32),
        ],
    )
    def build_adj(src_hbm, dst_hbm, a_hbm, acc, srcb, dstb):
        wid = lax.axis_index("s") * _NUM_CORES + lax.axis_index("c")
        zeros = jnp.zeros((_LANES,), dtype=jnp.int32)
        row0 = wid * rows

        for r in range(rows + 1):
            def zrow(i, _, r=r):
                for u in range(zunroll):
                    acc[r, pl.ds((i * zunroll + u) * _LANES, _LANES)] = zeros
                return 0
            lax.fori_loop(0, wcols // (_LANES * zunroll), zrow, 0)

        def chunk_body(c, _):
            pltpu.sync_copy(src_hbm.at[pl.ds(c * ech, ech)], srcb)
            pltpu.sync_copy(dst_hbm.at[pl.ds(c * ech, ech)], dstb)

            def edge_body(j, _):
                for u in range(eunroll):
                    o = (j * eunroll + u) * _LANES
                    sv = srcb[pl.ds(o, _LANES)]
                    dv = dstb[pl.ds(o, _LANES)]
                    rel = dv - row0
                    inr = rel.astype(jnp.uint32) < jnp.uint32(rows)
                    prow = ((rel & 1) << 5) + lax.shift_right_logical(rel, 1)
                    row = jnp.where(inr, prow, rows)
                    wcol = lax.shift_right_logical(sv, 1)
                    add = jnp.where((sv & 1) == 1, 65536, 1)
                    plsc.addupdate_scatter(acc, [row, wcol], add)
                return 0

            lax.fori_loop(0, ech // (_LANES * eunroll), edge_body, 0)
            return 0

        lax.fori_loop(0, n_chunks, chunk_body, 0)
        erow = pl.multiple_of(wid * half, 8)
        pltpu.sync_copy(acc.at[pl.ds(0, half)], a_hbm.at[pl.ds(erow, half)])
        pltpu.sync_copy(acc.at[pl.ds(half, half)],
                        a_hbm.at[pl.ds(nhalf + erow, half)])

    return build_adj


def _agg_kernel(a_ref, x_ref, ws_ref, wn_ref, bias_ref, g4_ref, out_ref,
                ab_ref, invd_ref):
    t = pl.program_id(0)
    bb = out_ref.shape[0]
    cc = out_ref.shape[3]
    nh = a_ref.shape[1]

    @pl.when(t == 0)
    def _prep():
        p = a_ref[...]
        hi = lax.shift_right_arithmetic(p, 16)
        lo = p - lax.shift_left(hi, 16)
        abe = lo.astype(jnp.float32).astype(jnp.bfloat16)
        abo = hi.astype(jnp.float32).astype(jnp.bfloat16)
        ab_ref[:, :nh] = abe
        ab_ref[:, nh:] = abo
        deg = lax.dot_general(
            abe + abo, jnp.ones((nh, 8), jnp.bfloat16),
            (((1,), (0,)), ((), ())), preferred_element_type=jnp.float32)
        invd_ref[...] = 1.0 / jnp.maximum(deg[:, :1], 1.0)

    x = x_ref[0]
    agg = lax.dot_general(ab_ref[...], x, (((1,), (0,)), ((), ())),
                          preferred_element_type=jnp.float32)
    s = (agg * invd_ref[...]).astype(jnp.bfloat16)
    hs = lax.dot_general(x, ws_ref[...], (((1,), (0,)), ((), ())),
                         preferred_element_type=jnp.float32)
    hn = lax.dot_general(s, wn_ref[...], (((1,), (0,)), ((), ())),
                         preferred_element_type=jnp.float32)
    h = hs + hn + bias_ref[...]
    n2 = lax.dot_general(h * h, g4_ref[...], (((1,), (0,)), ((), ())),
                         preferred_element_type=jnp.float32)
    r = 1.0 / jnp.maximum(jnp.sqrt(n2), 1e-12)
    for b in range(bb):
        out_ref[b, 0] = jnp.maximum(h[:, b * cc:(b + 1) * cc] * r[:, b:b + 1],
                                    0.0)


def kernel(blocks, node_feats, edge_feats, W_self, W_neigh, b):
    del edge_feats
    bn, nn, tn, cin = node_feats.shape
    cout = W_self.shape[1]
    en = blocks.shape[1]
    bc = bn * cout
    nh = nn // 2

    src = blocks[0].astype(jnp.int32)
    dst = blocks[1].astype(jnp.int32)
    adj = _build_adj_fn(nn, en)(src, dst)

    xs = jnp.transpose(node_feats.reshape(bn, nh, 2, tn, cin),
                       (3, 2, 1, 0, 4)).reshape(tn, nn, bc)
    xs = xs.astype(jnp.bfloat16)
    eye_b = jnp.eye(bn, dtype=jnp.float32)
    ws4 = jnp.kron(eye_b, W_self).astype(jnp.bfloat16)
    wn4 = jnp.kron(eye_b, W_neigh).astype(jnp.bfloat16)
    g4 = jnp.kron(eye_b, jnp.ones((cout, 1), jnp.float32))
    bias_row = jnp.tile(b, bn)[None, :]

    h = pl.pallas_call(
        _agg_kernel,
        grid=(tn,),
        in_specs=[
            pl.BlockSpec((nn, nh), lambda t: (0, 0)),
            pl.BlockSpec((1, nn, bc), lambda t: (t, 0, 0)),
            pl.BlockSpec((bc, bc), lambda t: (0, 0)),
            pl.BlockSpec((bc, bc), lambda t: (0, 0)),
            pl.BlockSpec((1, bc), lambda t: (0, 0)),
            pl.BlockSpec((bc, bn), lambda t: (0, 0)),
        ],
        out_specs=pl.BlockSpec((bn, 1, nn, cout), lambda t: (0, t, 0, 0)),
        out_shape=jax.ShapeDtypeStruct((bn, tn, nn, cout), jnp.float32),
        scratch_shapes=[
            pltpu.VMEM((nn, nn), jnp.bfloat16),
            pltpu.VMEM((nn, 1), jnp.float32),
        ],
    )(adj, xs, ws4, wn4, bias_row, g4)

    return jnp.transpose(h.reshape(bn, tn, 2, nh, cout),
                         (0, 3, 2, 1, 4)).reshape(bn, nn, tn, cout)


# speedup vs baseline: 1.3685x; 1.3685x over previous
import functools

import jax
import jax.numpy as jnp
from jax import lax
from jax.experimental import pallas as pl
from jax.experimental.pallas import tpu as pltpu
from jax.experimental.pallas import tpu_sc as plsc


_NUM_CORES = 2
_NUM_SUBCORES = 16
_LANES = 16


@functools.lru_cache(maxsize=None)
def _build_adj_fn(n_nodes: int, n_edges: int):
    n_workers = _NUM_CORES * _NUM_SUBCORES
    rows = n_nodes // n_workers
    wcols = n_nodes // 2
    ech = 16384
    n_chunks = n_edges // ech
    zunroll = 8
    eunroll = 4

    mesh = plsc.VectorSubcoreMesh(core_axis_name="c", subcore_axis_name="s")

    @functools.partial(
        pl.kernel,
        mesh=mesh,
        compiler_params=pltpu.CompilerParams(needs_layout_passes=False),
        out_type=jax.ShapeDtypeStruct((n_nodes, wcols), jnp.int32),
        scratch_types=[
            pltpu.VMEM((rows + 1, wcols), jnp.int32),
            pltpu.VMEM((ech,), jnp.int32),
            pltpu.VMEM((ech,), jnp.int32),
        ],
    )
    def build_adj(src_hbm, dst_hbm, a_hbm, acc, srcb, dstb):
        wid = lax.axis_index("s") * _NUM_CORES + lax.axis_index("c")
        zeros = jnp.zeros((_LANES,), dtype=jnp.int32)
        row0 = wid * rows

        for r in range(rows + 1):
            def zrow(i, _, r=r):
                for u in range(zunroll):
                    acc[r, pl.ds((i * zunroll + u) * _LANES, _LANES)] = zeros
                return 0
            lax.fori_loop(0, wcols // (_LANES * zunroll), zrow, 0)

        def chunk_body(c, _):
            pltpu.sync_copy(src_hbm.at[pl.ds(c * ech, ech)], srcb)
            pltpu.sync_copy(dst_hbm.at[pl.ds(c * ech, ech)], dstb)

            def edge_body(j, _):
                for u in range(eunroll):
                    o = (j * eunroll + u) * _LANES
                    sv = srcb[pl.ds(o, _LANES)]
                    dv = dstb[pl.ds(o, _LANES)]
                    rel = dv - row0
                    inr = rel.astype(jnp.uint32) < jnp.uint32(rows)
                    row = jnp.where(inr, rel, rows)
                    wcol = sv & (wcols - 1)
                    add = jnp.where(sv < wcols, 1, 65536)
                    plsc.addupdate_scatter(acc, [row, wcol], add)
                return 0

            lax.fori_loop(0, ech // (_LANES * eunroll), edge_body, 0)
            return 0

        lax.fori_loop(0, n_chunks, chunk_body, 0)
        pltpu.sync_copy(acc.at[pl.ds(0, rows)], a_hbm.at[pl.ds(row0, rows)])

    return build_adj


def _agg_kernel(a_ref, x_ref, ws_ref, wn_ref, bias_ref, g4_ref, out_ref,
                ab_ref, invd_ref):
    t = pl.program_id(0)
    bb = out_ref.shape[0]
    cc = out_ref.shape[3]
    nh = a_ref.shape[1]

    @pl.when(t == 0)
    def _prep():
        p = a_ref[...]
        hi = lax.shift_right_arithmetic(p, 16)
        lo = p - lax.shift_left(hi, 16)
        abe = lo.astype(jnp.float32).astype(jnp.bfloat16)
        abo = hi.astype(jnp.float32).astype(jnp.bfloat16)
        ab_ref[:, :nh] = abe
        ab_ref[:, nh:] = abo
        deg = lax.dot_general(
            abe + abo, jnp.ones((nh, 8), jnp.bfloat16),
            (((1,), (0,)), ((), ())), preferred_element_type=jnp.float32)
        invd_ref[...] = 1.0 / jnp.maximum(deg[:, :1], 1.0)

    x = x_ref[0]
    agg = lax.dot_general(ab_ref[...], x, (((1,), (0,)), ((), ())),
                          preferred_element_type=jnp.float32)
    s = (agg * invd_ref[...]).astype(jnp.bfloat16)
    hs = lax.dot_general(x, ws_ref[...], (((1,), (0,)), ((), ())),
                         preferred_element_type=jnp.float32)
    hn = lax.dot_general(s, wn_ref[...], (((1,), (0,)), ((), ())),
                         preferred_element_type=jnp.float32)
    h = hs + hn + bias_ref[...]
    n2 = lax.dot_general(h * h, g4_ref[...], (((1,), (0,)), ((), ())),
                         preferred_element_type=jnp.float32)
    r = 1.0 / jnp.maximum(jnp.sqrt(n2), 1e-12)
    for bi in range(bb):
        out_ref[bi, 0] = jnp.maximum(
            h[:, bi * cc:(bi + 1) * cc] * r[:, bi:bi + 1], 0.0)


def kernel(blocks, node_feats, edge_feats, W_self, W_neigh, b):
    del edge_feats
    bn, nn, tn, cin = node_feats.shape
    cout = W_self.shape[1]
    en = blocks.shape[1]
    bc = bn * cout
    nh = nn // 2

    src = blocks[0].astype(jnp.int32)
    dst = blocks[1].astype(jnp.int32)
    adj = _build_adj_fn(nn, en)(src, dst)

    xs = jnp.transpose(node_feats, (2, 1, 0, 3)).reshape(tn, nn, bc)
    xs = xs.astype(jnp.bfloat16)
    eye_b = jnp.eye(bn, dtype=jnp.float32)
    ws4 = jnp.kron(eye_b, W_self).astype(jnp.bfloat16)
    wn4 = jnp.kron(eye_b, W_neigh).astype(jnp.bfloat16)
    g4 = jnp.kron(eye_b, jnp.ones((cout, 1), jnp.float32))
    bias_row = jnp.tile(b, bn)[None, :]

    h = pl.pallas_call(
        _agg_kernel,
        grid=(tn,),
        in_specs=[
            pl.BlockSpec((nn, nh), lambda t: (0, 0)),
            pl.BlockSpec((1, nn, bc), lambda t: (t, 0, 0)),
            pl.BlockSpec((bc, bc), lambda t: (0, 0)),
            pl.BlockSpec((bc, bc), lambda t: (0, 0)),
            pl.BlockSpec((1, bc), lambda t: (0, 0)),
            pl.BlockSpec((bc, bn), lambda t: (0, 0)),
        ],
        out_specs=pl.BlockSpec((bn, 1, nn, cout), lambda t: (0, t, 0, 0)),
        out_shape=jax.ShapeDtypeStruct((bn, tn, nn, cout), jnp.float32),
        scratch_shapes=[
            pltpu.VMEM((nn, nn), jnp.bfloat16),
            pltpu.VMEM((nn, 1), jnp.float32),
        ],
    )(adj, xs, ws4, wn4, bias_row, g4)

    return jnp.transpose(h, (0, 2, 1, 3))


# speedup vs baseline: 1.4063x; 1.0277x over previous
import functools

import jax
import jax.numpy as jnp
from jax import lax
from jax.experimental import pallas as pl
from jax.experimental.pallas import tpu as pltpu
from jax.experimental.pallas import tpu_sc as plsc


_NUM_CORES = 2
_NUM_SUBCORES = 16
_LANES = 16


@functools.lru_cache(maxsize=None)
def _build_adj_fn(n_nodes: int, n_edges: int):
    n_workers = _NUM_CORES * _NUM_SUBCORES
    rows = n_nodes // n_workers
    wcols = n_nodes // 2
    ech = 16384
    n_chunks = n_edges // ech
    zunroll = 8
    eunroll = 8

    mesh = plsc.VectorSubcoreMesh(core_axis_name="c", subcore_axis_name="s")

    @functools.partial(
        pl.kernel,
        mesh=mesh,
        compiler_params=pltpu.CompilerParams(needs_layout_passes=False),
        out_type=jax.ShapeDtypeStruct((n_nodes, wcols), jnp.int32),
        scratch_types=[
            pltpu.VMEM((rows + 1, wcols), jnp.int32),
            pltpu.VMEM((ech,), jnp.int32),
            pltpu.VMEM((ech,), jnp.int32),
        ],
    )
    def build_adj(src_hbm, dst_hbm, a_hbm, acc, srcb, dstb):
        wid = lax.axis_index("s") * _NUM_CORES + lax.axis_index("c")
        zeros = jnp.zeros((_LANES,), dtype=jnp.int32)
        row0 = wid * rows

        for r in range(rows + 1):
            def zrow(i, _, r=r):
                for u in range(zunroll):
                    acc[r, pl.ds((i * zunroll + u) * _LANES, _LANES)] = zeros
                return 0
            lax.fori_loop(0, wcols // (_LANES * zunroll), zrow, 0)

        def chunk_body(c, _):
            pltpu.sync_copy(src_hbm.at[pl.ds(c * ech, ech)], srcb)
            pltpu.sync_copy(dst_hbm.at[pl.ds(c * ech, ech)], dstb)

            def edge_body(j, _):
                for u in range(eunroll):
                    o = (j * eunroll + u) * _LANES
                    sv = srcb[pl.ds(o, _LANES)]
                    dv = dstb[pl.ds(o, _LANES)]
                    rel = dv - row0
                    inr = rel.astype(jnp.uint32) < jnp.uint32(rows)
                    row = jnp.where(inr, rel, rows)
                    wcol = sv & (wcols - 1)
                    add = jnp.where(sv < wcols, 1, 65536)
                    plsc.addupdate_scatter(acc, [row, wcol], add)
                return 0

            lax.fori_loop(0, ech // (_LANES * eunroll), edge_body, 0)
            return 0

        lax.fori_loop(0, n_chunks, chunk_body, 0)
        pltpu.sync_copy(acc.at[pl.ds(0, rows)], a_hbm.at[pl.ds(row0, rows)])

    return build_adj


def _agg_kernel(a_ref, x_ref, ws_ref, wn_ref, bias_ref, g4_ref, g4t_ref,
                out_ref, ab_ref, invd_ref, o_ref):
    t = pl.program_id(0)
    nt = pl.num_programs(0) - 1
    bb = out_ref.shape[0]
    cc = out_ref.shape[3]
    nh = a_ref.shape[1]

    @pl.when(t == 0)
    def _prep():
        p = a_ref[...]
        hi = lax.shift_right_arithmetic(p, 16)
        lo = p - lax.shift_left(hi, 16)
        abe = lo.astype(jnp.float32).astype(jnp.bfloat16)
        abo = hi.astype(jnp.float32).astype(jnp.bfloat16)
        ab_ref[:, :nh] = abe
        ab_ref[:, nh:] = abo
        deg = lax.dot_general(
            abe + abo, jnp.ones((nh, 8), jnp.bfloat16),
            (((1,), (0,)), ((), ())), preferred_element_type=jnp.float32)
        invd_ref[...] = 1.0 / jnp.maximum(deg[:, :1], 1.0)

    @pl.when(t > 0)
    def _flush():
        for bi in range(bb):
            out_ref[bi, 0] = o_ref[:, bi * cc:(bi + 1) * cc]

    @pl.when(t < nt)
    def _compute():
        x = x_ref[0]
        agg = lax.dot_general(ab_ref[...], x, (((1,), (0,)), ((), ())),
                              preferred_element_type=jnp.float32)
        s = (agg * invd_ref[...]).astype(jnp.bfloat16)
        hs = lax.dot_general(x, ws_ref[...], (((1,), (0,)), ((), ())),
                             preferred_element_type=jnp.float32)
        hn = lax.dot_general(s, wn_ref[...], (((1,), (0,)), ((), ())),
                             preferred_element_type=jnp.float32)
        h = hs + hn + bias_ref[...]
        n2 = lax.dot_general(h * h, g4_ref[...], (((1,), (0,)), ((), ())),
                             preferred_element_type=jnp.float32)
        r = 1.0 / jnp.maximum(jnp.sqrt(n2), 1e-12)
        d = lax.dot_general(r, g4t_ref[...], (((1,), (0,)), ((), ())),
                            preferred_element_type=jnp.float32)
        o_ref[...] = jnp.maximum(h * d, 0.0)


def kernel(blocks, node_feats, edge_feats, W_self, W_neigh, b):
    del edge_feats
    bn, nn, tn, cin = node_feats.shape
    cout = W_self.shape[1]
    en = blocks.shape[1]
    bc = bn * cout
    nh = nn // 2

    src = blocks[0].astype(jnp.int32)
    dst = blocks[1].astype(jnp.int32)
    adj = _build_adj_fn(nn, en)(src, dst)

    xs = jnp.transpose(node_feats, (2, 1, 0, 3)).reshape(tn, nn, bc)
    xs = xs.astype(jnp.bfloat16)
    eye_b = jnp.eye(bn, dtype=jnp.float32)
    ws4 = jnp.kron(eye_b, W_self).astype(jnp.bfloat16)
    wn4 = jnp.kron(eye_b, W_neigh).astype(jnp.bfloat16)
    g4 = jnp.kron(eye_b, jnp.ones((cout, 1), jnp.float32))
    g4t = jnp.kron(eye_b, jnp.ones((1, cout), jnp.float32))
    bias_row = jnp.tile(b, bn)[None, :]

    tmax = tn - 1
    h = pl.pallas_call(
        _agg_kernel,
        grid=(tn + 1,),
        in_specs=[
            pl.BlockSpec((nn, nh), lambda t: (0, 0)),
            pl.BlockSpec((1, nn, bc), lambda t: (jnp.minimum(t, tmax), 0, 0)),
            pl.BlockSpec((bc, bc), lambda t: (0, 0)),
            pl.BlockSpec((bc, bc), lambda t: (0, 0)),
            pl.BlockSpec((1, bc), lambda t: (0, 0)),
            pl.BlockSpec((bc, bn), lambda t: (0, 0)),
            pl.BlockSpec((bn, bc), lambda t: (0, 0)),
        ],
        out_specs=pl.BlockSpec(
            (bn, 1, nn, cout),
            lambda t: (0, jnp.maximum(t, 1) - 1, 0, 0)),
        out_shape=jax.ShapeDtypeStruct((bn, tn, nn, cout), jnp.float32),
        scratch_shapes=[
            pltpu.VMEM((nn, nn), jnp.bfloat16),
            pltpu.VMEM((nn, 1), jnp.float32),
            pltpu.VMEM((nn, bc), jnp.float32),
        ],
    )(adj, xs, ws4, wn4, bias_row, g4, g4t)

    return jnp.transpose(h, (0, 2, 1, 3))
